# Initial kernel scaffold; baseline (speedup 1.0000x reference)
import jax
import jax.numpy as jnp
from jax.experimental import pallas as pl


def kernel(t, beta, alpha):
    raise NotImplementedError("write your pallas kernel here")


# speedup vs baseline: 6.1344x; 6.1344x over previous
import functools

import jax
import jax.numpy as jnp
from jax import lax
from jax.experimental import pallas as pl
from jax.experimental.pallas import tpu as pltpu
from jax.experimental.pallas import tpu_sc as plsc

_BATCH = 16384
_CHUNK = 128


@functools.cache
def _build_kernel():
    info = plsc.get_sparse_core_info()
    num_cores, num_subcores = info.num_cores, info.num_subcores
    num_workers = num_cores * num_subcores
    b_per_w = _BATCH // num_workers
    n_chunks = b_per_w // _CHUNK

    mesh = plsc.VectorSubcoreMesh(core_axis_name="c", subcore_axis_name="s")

    @functools.partial(
        pl.kernel,
        mesh=mesh,
        out_type=(
            jax.ShapeDtypeStruct((num_workers, n_chunks, _CHUNK), jnp.float32),
            jax.ShapeDtypeStruct((num_workers, n_chunks, _CHUNK), jnp.float32),
        ),
        scratch_types=[
            pltpu.VMEM((n_chunks, _CHUNK), jnp.int32),
            pltpu.VMEM((n_chunks, _CHUNK), jnp.float32),
            pltpu.VMEM((n_chunks, _CHUNK), jnp.float32),
            pltpu.SemaphoreType.DMA,
        ],
    )
    def ddpm_lookup(
        t_hbm,
        beta_hbm,
        alpha_hbm,
        beta_out_hbm,
        alpha_out_hbm,
        idx_v,
        beta_o_v,
        alpha_o_v,
        sem,
    ):
        wid = lax.axis_index("s") * num_cores + lax.axis_index("c")
        pltpu.sync_copy(t_hbm.at[wid], idx_v)
        descs = []
        for j in range(n_chunks):
            descs.append(
                pltpu.async_copy(beta_hbm.at[idx_v.at[j]], beta_o_v.at[j], sem)
            )
            descs.append(
                pltpu.async_copy(alpha_hbm.at[idx_v.at[j]], alpha_o_v.at[j], sem)
            )
        for d in descs:
            d.wait()
        pltpu.sync_copy(beta_o_v, beta_out_hbm.at[wid])
        pltpu.sync_copy(alpha_o_v, alpha_out_hbm.at[wid])

    return ddpm_lookup, num_workers, n_chunks


def kernel(t, beta, alpha):
    fn, num_workers, n_chunks = _build_kernel()
    t3 = t.astype(jnp.int32).reshape(num_workers, n_chunks, _CHUNK)
    beta_t, alpha_t = fn(t3, beta, alpha)
    return beta_t.reshape(_BATCH), alpha_t.reshape(_BATCH)
